# baseline (speedup 1.0000x reference)
import jax
import jax.numpy as jnp
from jax import lax
from jax.experimental import pallas as pl
from jax.experimental.pallas import tpu as pltpu
from jax.experimental.pallas import tpu_sc as plsc

VOCAB = 1000000
N_EMBD = 64
BATCH = 16384
SEQ = 50

_info = plsc.get_sparse_core_info()
NC = _info.num_cores
NS = _info.num_subcores
NW = NC * NS

BW_ = BATCH // NW
CB = 128
TPW = BW_ // CB
N_CHUNKS = SEQ * TPW
N_BLOCKS = SEQ * 8 * (BATCH // 128)

TC_GRID = -(-VOCAB // 256)
OUT_R = 128 * TC_GRID
TAB_R = 2 * OUT_R


def _tc_format_body(tin_ref, out_ref):
    x = tin_ref[...]
    out_ref[...] = jnp.concatenate(
        [jnp.transpose(x[:, :128]), jnp.transpose(x[:, 128:])], axis=1)


def _format_table(wte):
    tin = wte.T
    t2 = pl.pallas_call(
        _tc_format_body,
        grid=(TC_GRID,),
        in_specs=[pl.BlockSpec((N_EMBD, 256), lambda i: (0, i))],
        out_specs=pl.BlockSpec((128, 128), lambda i: (i, 0)),
        out_shape=jax.ShapeDtypeStruct((OUT_R, 128), jnp.float32),
    )(tin)
    return t2.reshape(TAB_R, N_EMBD)


def _body(ids_hbm, table_hbm, out_hbm,
          idx0, idx1, ih0, ih1, g0, g1, t0_, t1_,
          si0, si1, sg0, sg1, sw0, sw1):
    idxb = (idx0, idx1)
    ihb = (ih0, ih1)
    gb = (g0, g1)
    tb = (t0_, t1_)
    si = (si0, si1)
    sg = (sg0, sg1)
    sw = (sw0, sw1)

    wid = lax.axis_index("s") * NC + lax.axis_index("c")
    wb0 = wid * BW_

    iota = lax.iota(jnp.int32, 16)
    rvs = [iota + 16 * t0 for t0 in range(8)]

    def idx_off(k):
        return (k // TPW) * BATCH + wb0 + (k % TPW) * CB

    def idx_start(k, b):
        pltpu.async_copy(ids_hbm.at[pl.ds(idx_off(k), CB)], idxb[b], si[b])

    def idx_wait(b):
        pltpu.make_async_copy(ids_hbm.at[pl.ds(0, CB)], idxb[b], si[b]).wait()

    def perm_idx(b):
        for t0 in range(8):
            v = idxb[b][pl.ds(t0 * 16, 16)]
            ihb[b][pl.ds(t0 * 16, 16)] = (
                (v & ~jnp.int32(255)) + ((v & 127) << 1) + ((v >> 7) & 1))

    def gather_start(b):
        pltpu.async_copy(table_hbm.at[ihb[b]], gb[b], sg[b])

    def gather_wait(b):
        pltpu.make_async_copy(table_hbm.at[ihb[b]], gb[b], sg[b]).wait()

    def writes_start(k, b):
        s = k // TPW
        j = wid * TPW + (k % TPW)
        for i in range(8):
            blk = (s * 8 + i) * 128 + j
            pltpu.async_copy(tb[b].at[i], out_hbm.at[blk], sw[b])

    def writes_wait(b):
        for i in range(8):
            pltpu.make_async_copy(tb[b].at[i], out_hbm.at[0], sw[b]).wait()

    def transpose(b):
        G = gb[b]
        T = tb[b]

        @plsc.parallel_loop(0, N_EMBD, 1, unroll=4)
        def _(e):
            evec = lax.broadcast_in_dim(e, (16,), ())
            f = (evec + iota) & 63
            fi = f >> 3
            fr = f & 7
            for t0 in range(8):
                v = plsc.load_gather(G, [rvs[t0], f])
                plsc.store_scatter(T, [fi, fr, rvs[t0]], v)

    idx_start(0, 0)
    idx_start(1, 1)
    idx_wait(0)
    perm_idx(0)
    gather_start(0)

    def chunk(k, b):
        gather_wait(b)

        nb = b ^ 1

        @pl.when(k + 1 < N_CHUNKS)
        def _():
            idx_wait(nb)
            perm_idx(nb)
            gather_start(nb)

        @pl.when(k + 2 < N_CHUNKS)
        def _():
            idx_start(k + 2, b)

        @pl.when(k >= 2)
        def _():
            writes_wait(b)

        transpose(b)
        writes_start(k, b)

    def outer(o, carry):
        k = o * 2
        chunk(k, 0)
        chunk(k + 1, 1)
        return carry

    lax.fori_loop(0, N_CHUNKS // 2, outer, 0, unroll=False)

    writes_wait(0)
    writes_wait(1)


@jax.jit
def kernel(input_ids, wte):
    ids_flat = input_ids.T.reshape(-1)
    table = _format_table(wte)
    mesh = plsc.VectorSubcoreMesh(core_axis_name="c", subcore_axis_name="s")
    out3 = pl.kernel(
        _body,
        out_type=jax.ShapeDtypeStruct((N_BLOCKS, 8, 128), jnp.float32),
        mesh=mesh,
        scratch_types=(
            [pltpu.VMEM((CB,), jnp.int32) for _ in range(4)]
            + [pltpu.VMEM((CB, N_EMBD), jnp.float32) for _ in range(2)]
            + [pltpu.VMEM((8, 8, 128), jnp.float32) for _ in range(2)]
            + [pltpu.SemaphoreType.DMA for _ in range(6)]
        ),
        compiler_params=pltpu.CompilerParams(use_tc_tiling_on_sc=False,
                                             needs_layout_passes=False),
    )(ids_flat, table)
    X = out3.reshape(SEQ, 8, 128, 8, 128)
    return X.transpose(2, 4, 0, 1, 3).reshape(BATCH, SEQ, N_EMBD)


# speedup vs baseline: 2.5278x; 2.5278x over previous
import jax
import jax.numpy as jnp
from jax import lax
from jax.experimental import pallas as pl
from jax.experimental.pallas import tpu as pltpu
from jax.experimental.pallas import tpu_sc as plsc

VOCAB = 1000000
N_EMBD = 64
BATCH = 16384
SEQ = 50

_info = plsc.get_sparse_core_info()
NC = _info.num_cores
NS = _info.num_subcores
NW = NC * NS

BW_ = BATCH // NW
CB = 128
TPW = BW_ // CB
N_CHUNKS = SEQ * TPW
N_BLOCKS = SEQ * 8 * (BATCH // 128)

TC_GRID = -(-VOCAB // 1024)
OUT_R = 512 * TC_GRID
TAB_R = 2 * OUT_R


def _tc_format_body(tin_ref, out_ref):
    x = tin_ref[...]
    r = lax.broadcasted_iota(jnp.int32, (N_EMBD, N_EMBD), 0)
    c = lax.broadcasted_iota(jnp.int32, (N_EMBD, N_EMBD), 1)
    eye = (r == c).astype(jnp.float32)
    rows = []
    for q in range(4):
        halves = []
        for j in range(2):
            xs = x[:, 256 * q + 128 * j: 256 * q + 128 * j + 128]
            halves.append(lax.dot_general(
                xs, eye, (((0,), (0,)), ((), ())),
                preferred_element_type=jnp.float32))
        rows.append(jnp.concatenate(halves, axis=1))
    out_ref[...] = jnp.concatenate(rows, axis=0)


def _format_table(wte):
    tin = wte.T
    t2 = pl.pallas_call(
        _tc_format_body,
        grid=(TC_GRID,),
        in_specs=[pl.BlockSpec((N_EMBD, 1024), lambda i: (0, i))],
        out_specs=pl.BlockSpec((512, 128), lambda i: (i, 0)),
        out_shape=jax.ShapeDtypeStruct((OUT_R, 128), jnp.float32),
    )(tin)
    return t2.reshape(TAB_R, N_EMBD)


def _body(ids_hbm, table_hbm, out_hbm,
          idx0, idx1, ih0, ih1, g0, g1, t0_, t1_,
          si0, si1, sg0, sg1, sw0, sw1):
    idxb = (idx0, idx1)
    ihb = (ih0, ih1)
    gb = (g0, g1)
    tb = (t0_, t1_)
    si = (si0, si1)
    sg = (sg0, sg1)
    sw = (sw0, sw1)

    wid = lax.axis_index("s") * NC + lax.axis_index("c")
    wb0 = wid * BW_

    iota = lax.iota(jnp.int32, 16)
    rvs = [iota + 16 * t0 for t0 in range(8)]

    def idx_off(k):
        return (k // TPW) * BATCH + wb0 + (k % TPW) * CB

    def idx_start(k, b):
        pltpu.async_copy(ids_hbm.at[pl.ds(idx_off(k), CB)], idxb[b], si[b])

    def idx_wait(b):
        pltpu.make_async_copy(ids_hbm.at[pl.ds(0, CB)], idxb[b], si[b]).wait()

    def perm_idx(b):
        for t0 in range(8):
            v = idxb[b][pl.ds(t0 * 16, 16)]
            ihb[b][pl.ds(t0 * 16, 16)] = (
                (v & ~jnp.int32(255)) + ((v & 127) << 1) + ((v >> 7) & 1))

    def gather_start(b):
        pltpu.async_copy(table_hbm.at[ihb[b]], gb[b], sg[b])

    def gather_wait(b):
        pltpu.make_async_copy(table_hbm.at[ihb[b]], gb[b], sg[b]).wait()

    def writes_start(k, b):
        s = k // TPW
        j = wid * TPW + (k % TPW)
        for i in range(8):
            blk = (s * 8 + i) * 128 + j
            pltpu.async_copy(tb[b].at[i], out_hbm.at[blk], sw[b])

    def writes_wait(b):
        for i in range(8):
            pltpu.make_async_copy(tb[b].at[i], out_hbm.at[0], sw[b]).wait()

    def transpose(b):
        G = gb[b]
        T = tb[b]

        @plsc.parallel_loop(0, N_EMBD, 1, unroll=4)
        def _(e):
            evec = lax.broadcast_in_dim(e, (16,), ())
            f = (evec + iota) & 63
            fi = f >> 3
            fr = f & 7
            for t0 in range(8):
                v = plsc.load_gather(G, [rvs[t0], f])
                plsc.store_scatter(T, [fi, fr, rvs[t0]], v)

    idx_start(0, 0)
    idx_start(1, 1)
    idx_wait(0)
    perm_idx(0)
    gather_start(0)

    def chunk(k, b):
        gather_wait(b)

        nb = b ^ 1

        @pl.when(k + 1 < N_CHUNKS)
        def _():
            idx_wait(nb)
            perm_idx(nb)
            gather_start(nb)

        @pl.when(k + 2 < N_CHUNKS)
        def _():
            idx_start(k + 2, b)

        @pl.when(k >= 2)
        def _():
            writes_wait(b)

        transpose(b)
        writes_start(k, b)

    def outer(o, carry):
        k = o * 2
        chunk(k, 0)
        chunk(k + 1, 1)
        return carry

    lax.fori_loop(0, N_CHUNKS // 2, outer, 0, unroll=False)

    writes_wait(0)
    writes_wait(1)


@jax.jit
def kernel(input_ids, wte):
    ids_flat = input_ids.T.reshape(-1)
    table = _format_table(wte)
    mesh = plsc.VectorSubcoreMesh(core_axis_name="c", subcore_axis_name="s")
    out3 = pl.kernel(
        _body,
        out_type=jax.ShapeDtypeStruct((N_BLOCKS, 8, 128), jnp.float32),
        mesh=mesh,
        scratch_types=(
            [pltpu.VMEM((CB,), jnp.int32) for _ in range(4)]
            + [pltpu.VMEM((CB, N_EMBD), jnp.float32) for _ in range(2)]
            + [pltpu.VMEM((8, 8, 128), jnp.float32) for _ in range(2)]
            + [pltpu.SemaphoreType.DMA for _ in range(6)]
        ),
        compiler_params=pltpu.CompilerParams(use_tc_tiling_on_sc=False,
                                             needs_layout_passes=False),
    )(ids_flat, table)
    X = out3.reshape(SEQ, 8, 128, 8, 128)
    return X.transpose(2, 4, 0, 1, 3).reshape(BATCH, SEQ, N_EMBD)


# speedup vs baseline: 2.9139x; 1.1527x over previous
import jax
import jax.numpy as jnp
from jax import lax
from jax.experimental import pallas as pl
from jax.experimental.pallas import tpu as pltpu
from jax.experimental.pallas import tpu_sc as plsc

VOCAB = 1000000
N_EMBD = 64
BATCH = 16384
SEQ = 50

_info = plsc.get_sparse_core_info()
NC = _info.num_cores
NS = _info.num_subcores
NW = NC * NS

BW_ = BATCH // NW
CB = 128
TPW = BW_ // CB
N_CHUNKS = SEQ * TPW
N_BLOCKS = SEQ * 8 * (BATCH // 128)

def _body(ids_hbm, table_hbm, out_hbm,
          idx0, idx1, g0, g1, t0_, t1_,
          si0, si1, sg0, sg1, sw0, sw1):
    idxb = (idx0, idx1)
    gb = (g0, g1)
    tb = (t0_, t1_)
    si = (si0, si1)
    sg = (sg0, sg1)
    sw = (sw0, sw1)

    wid = lax.axis_index("s") * NC + lax.axis_index("c")
    wb0 = wid * BW_

    iota = lax.iota(jnp.int32, 16)
    rvs = [iota + 16 * t0 for t0 in range(8)]

    def idx_off(k):
        return (k // TPW) * BATCH + wb0 + (k % TPW) * CB

    def idx_start(k, b):
        pltpu.async_copy(ids_hbm.at[pl.ds(idx_off(k), CB)], idxb[b], si[b])

    def idx_wait(b):
        pltpu.make_async_copy(ids_hbm.at[pl.ds(0, CB)], idxb[b], si[b]).wait()

    def gather_start(b):
        pltpu.async_copy(table_hbm.at[idxb[b]], gb[b], sg[b])

    def gather_wait(b):
        pltpu.make_async_copy(table_hbm.at[idxb[b]], gb[b], sg[b]).wait()

    def writes_start(k, b):
        s = k // TPW
        j = wid * TPW + (k % TPW)
        for i in range(8):
            blk = (s * 8 + i) * 128 + j
            pltpu.async_copy(tb[b].at[i], out_hbm.at[blk], sw[b])

    def writes_wait(b):
        for i in range(8):
            pltpu.make_async_copy(tb[b].at[i], out_hbm.at[0], sw[b]).wait()

    def transpose(b):
        G = gb[b]
        T = tb[b]

        @plsc.parallel_loop(0, N_EMBD, 1, unroll=4)
        def _(e):
            evec = lax.broadcast_in_dim(e, (16,), ())
            f = (evec + iota) & 63
            fi = f >> 3
            fr = f & 7
            for t0 in range(8):
                v = plsc.load_gather(G, [rvs[t0], f])
                plsc.store_scatter(T, [fi, fr, rvs[t0]], v)

    idx_start(0, 0)
    idx_start(1, 1)
    idx_wait(0)
    gather_start(0)

    def chunk(k, b):
        gather_wait(b)

        nb = b ^ 1

        @pl.when(k + 1 < N_CHUNKS)
        def _():
            idx_wait(nb)
            gather_start(nb)

        @pl.when(k + 2 < N_CHUNKS)
        def _():
            idx_start(k + 2, b)

        @pl.when(k >= 2)
        def _():
            writes_wait(b)

        transpose(b)
        writes_start(k, b)

    def outer(o, carry):
        k = o * 2
        chunk(k, 0)
        chunk(k + 1, 1)
        return carry

    lax.fori_loop(0, N_CHUNKS // 2, outer, 0, unroll=False)

    writes_wait(0)
    writes_wait(1)


@jax.jit
def kernel(input_ids, wte):
    ids_flat = input_ids.T.reshape(-1)
    mesh = plsc.VectorSubcoreMesh(core_axis_name="c", subcore_axis_name="s")
    out3 = pl.kernel(
        _body,
        out_type=jax.ShapeDtypeStruct((N_BLOCKS, 8, 128), jnp.float32),
        mesh=mesh,
        scratch_types=(
            [pltpu.VMEM((CB,), jnp.int32) for _ in range(2)]
            + [pltpu.VMEM((CB, N_EMBD), jnp.float32) for _ in range(2)]
            + [pltpu.VMEM((8, 8, 128), jnp.float32) for _ in range(2)]
            + [pltpu.SemaphoreType.DMA for _ in range(6)]
        ),
        compiler_params=pltpu.CompilerParams(use_tc_tiling_on_sc=False,
                                             needs_layout_passes=False),
    )(ids_flat, wte)
    X = out3.reshape(SEQ, 8, 128, 8, 128)
    return X.transpose(2, 4, 0, 1, 3).reshape(BATCH, SEQ, N_EMBD)
